# baseline (speedup 1.0000x reference)
import functools

import jax
import jax.numpy as jnp
from jax import lax
from jax.experimental import pallas as pl
from jax.experimental.pallas import tpu as pltpu
from jax.experimental.pallas import tpu_sc as plsc

N = 10000
E = 320000
NP = 10240
NC = 2
NS = 16
NW = NC * NS
K = 80
EPW = 10000
C = E // (NW * K)
DG = 5
CG = C // DG
RPT = NP // NS

_MESH = plsc.VectorSubcoreMesh(core_axis_name="c", subcore_axis_name="s")


def _zero_rows(ref, nrows, ncols):
  z16 = jnp.zeros((16,), jnp.float32)

  def body(r, carry):
    for cc in range(ncols // 16):
      ref[r, pl.ds(cc * 16, 16)] = z16
    return carry

  lax.fori_loop(0, nrows, body, 0)


def _deg_kernel(dst_hbm, degp_hbm, ones_v, dst_v, zb_v, shared):
  c = lax.axis_index("c")
  s = lax.axis_index("s")
  wid = c * NS + s

  one16 = jnp.ones((16,), jnp.float32)
  z16 = jnp.zeros((16,), jnp.float32)
  for i in range(K // 16):
    ones_v[pl.ds(16 * i, 16)] = one16
  if K % 16:
    ones_v[pl.ds(K - 16, 16)] = one16
  for i in range(RPT // 16):
    zb_v[pl.ds(16 * i, 16)] = z16

  pltpu.sync_copy(zb_v, shared.at[pl.ds(s * RPT, RPT)])
  plsc.subcore_barrier()

  pltpu.sync_copy(dst_hbm.at[wid], dst_v)

  def body(j, carry):
    pltpu.sync_copy(ones_v, shared.at[dst_v.at[j]], add=True)
    return carry

  lax.fori_loop(0, C, body, 0)
  plsc.subcore_barrier()

  pltpu.sync_copy(shared.at[pl.ds(s * RPT, RPT)],
                  degp_hbm.at[c, pl.ds(s * RPT, RPT)])


def _make_deg():
  return functools.partial(
      pl.kernel,
      out_type=jax.ShapeDtypeStruct((NC, NP), jnp.float32),
      mesh=_MESH,
      scratch_types=[
          pltpu.VMEM((K,), jnp.float32),
          pltpu.VMEM((C, K), jnp.int32),
          pltpu.VMEM((RPT,), jnp.float32),
          pltpu.VMEM_SHARED((NP,), jnp.float32),
      ],
  )(_deg_kernel)


def _agg_kernel(d, src_hbm, dst_hbm, xs_hbm, out_hbm,
                src_v, dst_g, rows_a, rows_b, rows_c, shared,
                sem_a, sem_b, sem_c, sem_d):
  c = lax.axis_index("c")
  s = lax.axis_index("s")
  wid = c * NS + s

  _zero_rows(rows_a, K, d)
  for t in range(RPT // K):
    pltpu.sync_copy(rows_a, shared.at[pl.ds(s * RPT + t * K, K)])
  plsc.subcore_barrier()

  pltpu.sync_copy(src_hbm.at[wid], src_v)

  def gather(j, buf, sem):
    pltpu.async_copy(xs_hbm.at[src_v.at[pl.ds(j * K, K)]], buf, sem)

  def wait(j, buf, sem):
    pltpu.make_async_copy(xs_hbm.at[src_v.at[pl.ds(j * K, K)]], buf,
                          sem).wait()

  pltpu.sync_copy(dst_hbm.at[wid, 0], dst_g.at[0])
  pltpu.async_copy(dst_hbm.at[wid, 1], dst_g.at[1], sem_d)

  for g in range(DG):
    if g > 0:
      pltpu.make_async_copy(dst_hbm.at[wid, g], dst_g.at[g % 2],
                            sem_d).wait()
      if g + 1 < DG:
        pltpu.async_copy(dst_hbm.at[wid, g + 1], dst_g.at[(g + 1) % 2],
                         sem_d)
    jb = g * CG
    gslot = g % 2

    def scatter(lj, buf, gslot=gslot):
      pltpu.sync_copy(buf, shared.at[dst_g.at[gslot, lj]], add=True)

    gather(jb, rows_a, sem_a)
    gather(jb + 1, rows_b, sem_b)

    def triple(t, carry, jb=jb, scatter=scatter):
      lj = 3 * t
      j = jb + lj

      @pl.when(lj + 2 < CG)
      def _():
        gather(j + 2, rows_c, sem_c)

      wait(j, rows_a, sem_a)
      scatter(lj, rows_a)

      @pl.when(lj + 3 < CG)
      def _():
        gather(j + 3, rows_a, sem_a)

      @pl.when(lj + 1 < CG)
      def _():
        wait(j + 1, rows_b, sem_b)
        scatter(lj + 1, rows_b)

      @pl.when(lj + 4 < CG)
      def _():
        gather(j + 4, rows_b, sem_b)

      @pl.when(lj + 2 < CG)
      def _():
        wait(j + 2, rows_c, sem_c)
        scatter(lj + 2, rows_c)

      return carry

    lax.fori_loop(0, (CG + 2) // 3, triple, 0)

  plsc.subcore_barrier()

  pltpu.sync_copy(shared.at[pl.ds(s * RPT, RPT)],
                  out_hbm.at[c, pl.ds(s * RPT, RPT)])


def _make_agg(d):
  return functools.partial(
      pl.kernel,
      out_type=jax.ShapeDtypeStruct((NC, NP, d), jnp.float32),
      mesh=_MESH,
      scratch_types=[
          pltpu.VMEM((EPW,), jnp.int32),
          pltpu.VMEM((2, CG, K), jnp.int32),
          pltpu.VMEM((K, d), jnp.float32),
          pltpu.VMEM((K, d), jnp.float32),
          pltpu.VMEM((K, d), jnp.float32),
          pltpu.VMEM_SHARED((NP, d), jnp.float32),
          pltpu.SemaphoreType.DMA,
          pltpu.SemaphoreType.DMA,
          pltpu.SemaphoreType.DMA,
          pltpu.SemaphoreType.DMA,
      ],
  )(functools.partial(_agg_kernel, d))


_R = 400


def _dinv_block(degp_ref):
  blk = degp_ref[...]
  deg = blk[:, 0] + blk[:, 1] + 1.0
  return lax.rsqrt(deg)[:, None]


def _tc1_kernel(x_ref, w1_ref, degp_ref, xs_ref):
  h1 = jnp.dot(x_ref[...], w1_ref[...], preferred_element_type=jnp.float32)
  xs_ref[...] = _dinv_block(degp_ref) * h1


def _tc2_kernel(p_ref, xs_ref, degp_ref, w2_ref, b1_ref, ys_ref):
  dinv = _dinv_block(degp_ref)
  acc = p_ref[0] + p_ref[1] + xs_ref[...]
  h = jnp.maximum(dinv * acc + b1_ref[...], 0.0)
  h2 = jnp.dot(h, w2_ref[...], preferred_element_type=jnp.float32)
  ys_ref[...] = dinv * h2


def _tc3_kernel(d_out, q_ref, ys_ref, degp_ref, b2_ref, z_ref):
  dinv = _dinv_block(degp_ref)
  acc = q_ref[0] + q_ref[1] + ys_ref[...]
  z = jnp.maximum(dinv * acc + b2_ref[...], 0.0)
  z_ref[...] = z[:, :d_out]


def _row_blocked(d):
  return pl.BlockSpec((_R, d), lambda i: (i, 0))


def _degp_spec():
  return pl.BlockSpec((_R, 2), lambda i: (i, 0))


def _full(shape):
  return pl.BlockSpec(shape, lambda i: tuple(0 for _ in shape))


def kernel(x, edge_index, W1, b1, W2, b2):
  d_in = x.shape[1]
  d_hid = W1.shape[1]
  d_out = W2.shape[1]

  W2p = jnp.pad(W2, ((0, 0), (0, d_hid - d_out)))
  b2p = jnp.pad(b2, (0, d_hid - d_out))

  src = edge_index[0].reshape(NW, EPW)
  dst3 = edge_index[1].reshape(NW, C, K)
  dst = edge_index[1].reshape(NW, DG, CG, K)

  degp = _make_deg()(dst3)
  degp_t = degp.T

  xs = pl.pallas_call(
      _tc1_kernel,
      grid=(N // _R,),
      in_specs=[_row_blocked(d_in), _full((d_in, d_hid)), _degp_spec()],
      out_specs=_row_blocked(d_hid),
      out_shape=jax.ShapeDtypeStruct((N, d_hid), jnp.float32),
  )(x, W1, degp_t)

  p = _make_agg(d_hid)(src, dst, xs)

  ys = pl.pallas_call(
      _tc2_kernel,
      grid=(N // _R,),
      in_specs=[
          pl.BlockSpec((2, _R, d_hid), lambda i: (0, i, 0)),
          _row_blocked(d_hid),
          _degp_spec(),
          _full((d_hid, d_hid)),
          _full((1, d_hid)),
      ],
      out_specs=_row_blocked(d_hid),
      out_shape=jax.ShapeDtypeStruct((N, d_hid), jnp.float32),
  )(p, xs, degp_t, W2p, b1.reshape(1, d_hid))

  q = _make_agg(d_hid)(src, dst, ys)

  z = pl.pallas_call(
      functools.partial(_tc3_kernel, d_out),
      grid=(N // _R,),
      in_specs=[
          pl.BlockSpec((2, _R, d_hid), lambda i: (0, i, 0)),
          _row_blocked(d_hid),
          _degp_spec(),
          _full((1, d_hid)),
      ],
      out_specs=_row_blocked(d_out),
      out_shape=jax.ShapeDtypeStruct((N, d_out), jnp.float32),
  )(q, ys, degp_t, b2p.reshape(1, d_hid))

  return z
